# baseline (speedup 1.0000x reference)
import functools

import jax
import jax.numpy as jnp
from jax import lax
from jax.experimental import pallas as pl
from jax.experimental.pallas import tpu as pltpu
from jax.experimental.pallas import tpu_sc as plsc

NC = 2
NS = 16
NW = NC * NS
L = 16

B = 16384
D = 16
BPW = B // NW
KCH = 4
CHUNK = BPW // KCH


def _body(users_r, items_r, ut_r, it_r, out_r,
          uidx, iidx, urows, irows, outv, sem):
    w = lax.axis_index("s") * NC + lax.axis_index("c")
    base = w * BPW

    pltpu.sync_copy(users_r.at[pl.ds(w * KCH, KCH)], uidx)
    pltpu.sync_copy(items_r.at[pl.ds(w * KCH, KCH)], iidx)

    handles = []
    for k in range(KCH):
        handles.append(pltpu.async_copy(
            ut_r.at[uidx.at[k]], urows.at[pl.ds(k * CHUNK, CHUNK), :], sem))
        handles.append(pltpu.async_copy(
            it_r.at[iidx.at[k]], irows.at[pl.ds(k * CHUNK, CHUNK), :], sem))
    for h in handles:
        h.wait()

    iota = lax.iota(jnp.int32, L)

    def group(g, carry):
        rowbase = g * L
        row_idx = iota + rowbase
        acc = jnp.zeros((L,), jnp.float32)
        for d in range(D):
            dcol = jnp.full((L,), d, jnp.int32)
            u = plsc.load_gather(urows, [row_idx, dcol])
            v = plsc.load_gather(irows, [row_idx, dcol])
            acc = acc + u * v
        outv[pl.ds(rowbase, L)] = acc
        return carry

    lax.fori_loop(0, BPW // L, group, 0)

    pltpu.sync_copy(outv, out_r.at[pl.ds(base, BPW)])


@functools.partial(
    pl.kernel,
    out_type=jax.ShapeDtypeStruct((B,), jnp.float32),
    mesh=plsc.VectorSubcoreMesh(core_axis_name="c", subcore_axis_name="s"),
    compiler_params=pltpu.CompilerParams(
        needs_layout_passes=False, use_tc_tiling_on_sc=False),
    scratch_types=[
        pltpu.VMEM((KCH, CHUNK), jnp.int32),
        pltpu.VMEM((KCH, CHUNK), jnp.int32),
        pltpu.VMEM((BPW, D), jnp.float32),
        pltpu.VMEM((BPW, D), jnp.float32),
        pltpu.VMEM((BPW,), jnp.float32),
        pltpu.SemaphoreType.DMA,
    ],
)
def _sc_kernel(users_r, items_r, ut_r, it_r, out_r,
               uidx, iidx, urows, irows, outv, sem):
    _body(users_r, items_r, ut_r, it_r, out_r,
          uidx, iidx, urows, irows, outv, sem)


def kernel(users, items, users_table, items_table):
    users2d = users.astype(jnp.int32).reshape(NW * KCH, CHUNK)
    items2d = items.astype(jnp.int32).reshape(NW * KCH, CHUNK)
    out = _sc_kernel(users2d, items2d, users_table, items_table)
    return out.reshape(B, 1)


# speedup vs baseline: 1.0027x; 1.0027x over previous
import functools

import jax
import jax.numpy as jnp
from jax import lax
from jax.experimental import pallas as pl
from jax.experimental.pallas import tpu as pltpu
from jax.experimental.pallas import tpu_sc as plsc

NC = 2
NS = 16
NW = NC * NS
L = 16

B = 16384
D = 16
BPW = B // NW
IDXC = 128
KCH = BPW // IDXC


def _body(users_r, items_r, ut_r, it_r, out_r,
          uidx, iidx, urows, irows, outv, sem):
    w = lax.axis_index("s") * NC + lax.axis_index("c")
    base = w * BPW

    pltpu.sync_copy(users_r.at[pl.ds(base, BPW)], uidx)
    pltpu.sync_copy(items_r.at[pl.ds(base, BPW)], iidx)

    handles = []
    for k in range(KCH):
        sl = pl.ds(k * IDXC, IDXC)
        handles.append(pltpu.async_copy(
            ut_r.at[uidx.at[sl]], urows.at[pl.ds(k * IDXC, IDXC), :], sem))
        handles.append(pltpu.async_copy(
            it_r.at[iidx.at[sl]], irows.at[pl.ds(k * IDXC, IDXC), :], sem))
    for h in handles:
        h.wait()

    iota = lax.iota(jnp.int32, L)

    def group(g, carry):
        row_idx = iota + g * L
        acc = jnp.zeros((L,), jnp.float32)
        for d in range(D):
            dcol = jnp.full((L,), d, jnp.int32)
            u = plsc.load_gather(urows, [row_idx, dcol])
            v = plsc.load_gather(irows, [row_idx, dcol])
            acc = acc + u * v
        outv[pl.ds(g * L, L)] = acc
        return carry

    lax.fori_loop(0, BPW // L, group, 0)

    pltpu.sync_copy(outv, out_r.at[pl.ds(base, BPW)])


@functools.partial(
    pl.kernel,
    out_type=jax.ShapeDtypeStruct((B,), jnp.float32),
    mesh=plsc.VectorSubcoreMesh(core_axis_name="c", subcore_axis_name="s"),
    compiler_params=pltpu.CompilerParams(
        needs_layout_passes=False, use_tc_tiling_on_sc=False),
    scratch_types=[
        pltpu.VMEM((BPW,), jnp.int32),
        pltpu.VMEM((BPW,), jnp.int32),
        pltpu.VMEM((BPW, D), jnp.float32),
        pltpu.VMEM((BPW, D), jnp.float32),
        pltpu.VMEM((BPW,), jnp.float32),
        pltpu.SemaphoreType.DMA,
    ],
)
def _sc_kernel(users_r, items_r, ut_r, it_r, out_r,
               uidx, iidx, urows, irows, outv, sem):
    _body(users_r, items_r, ut_r, it_r, out_r,
          uidx, iidx, urows, irows, outv, sem)


def kernel(users, items, users_table, items_table):
    users = users.astype(jnp.int32)
    items = items.astype(jnp.int32)
    zero = (users[0] * 0).astype(jnp.float32)
    out = _sc_kernel(users, items, users_table + zero, items_table + zero)
    return out.reshape(B, 1)
